# baseline (speedup 1.0000x reference)
import functools

import jax
import jax.numpy as jnp
import numpy as np
from jax import lax
from jax.experimental import pallas as pl
from jax.experimental.pallas import tpu as pltpu
from jax.experimental.pallas import tpu_sc as plsc

_NCLS = 80
_ANCH = np.array(
    [[10., 13., 16., 30., 33., 23.],
     [30., 61., 62., 45., 59., 119.],
     [116., 90., 156., 198., 373., 326.]],
    dtype=np.float32,
).reshape(3, 3, 2)
_OFFS = [(0.0, 0.0), (0.5, 0.0), (0.0, 0.5), (-0.5, 0.0), (0.0, -0.5)]

_LVL = [(128, 80, 80), (256, 40, 40), (512, 20, 20)]
_B = 8
_NT = 300
_NCAND = 15 * _NT
_NPAD = 4512
_STR = 128

_SC_SPLIT = [(2, 800), (1, 400), (1, 100)]
_NTILES = 32


def _idx_body(tt_ref, q0, v0, q1, v1, q2, v2):
    tt = tt_ref[...]
    img, cls = tt[0:1], tt[1:2]
    x, y, w, h = tt[2:3], tt[3:4], tt[4:5], tt[5:6]
    b = img.astype(jnp.int32)
    c = cls.astype(jnp.int32)
    qrefs = (q0, q1, q2)
    vrefs = (v0, v1, v2)
    for l, (_, H, W) in enumerate(_LVL):
        gx, gy = x * W, y * H
        gw, gh = w * W, h * H
        jms = []
        for a in range(3):
            aw, ah = float(_ANCH[l, a, 0]), float(_ANCH[l, a, 1])
            rw, rh = gw / aw, gh / ah
            ratio = jnp.maximum(jnp.maximum(rw, 1.0 / rw),
                                jnp.maximum(rh, 1.0 / rh))
            jms.append(ratio < 4.0)
        gxi, gyi = W - gx, H - gy
        jj = ((gx % 1.0) < 0.5) & (gx > 1.0)
        kk = ((gy % 1.0) < 0.5) & (gy > 1.0)
        ll = ((gxi % 1.0) < 0.5) & (gxi > 1.0)
        mm = ((gyi % 1.0) < 0.5) & (gyi > 1.0)
        sels = [jnp.ones_like(jj), jj, kk, ll, mm]
        qrows, vrows = [], []
        for o in range(5):
            ox, oy = _OFFS[o]
            gi = jnp.clip((gx - ox).astype(jnp.int32), 0, W - 1)
            gj = jnp.clip((gy - oy).astype(jnp.int32), 0, H - 1)
            qo = (b * (H * W) + gj * W + gi) * _STR + c
            for a in range(3):
                qrows.append(qo)
                vrows.append((sels[o] & jms[a]).astype(jnp.float32))
        qrefs[l][...] = jnp.concatenate(qrows, axis=0)
        vrefs[l][...] = jnp.concatenate(vrows, axis=0)


def _build_indices(target):
    tt = jnp.transpose(target)
    outs = pl.pallas_call(
        _idx_body,
        out_shape=[
            s
            for _ in range(3)
            for s in (jax.ShapeDtypeStruct((15, _NT), jnp.int32),
                      jax.ShapeDtypeStruct((15, _NT), jnp.float32))
        ],
    )(tt)
    padded = []
    for arr in outs:
        flat = jnp.reshape(arr, (_NCAND,))
        flat = jnp.concatenate(
            [flat, jnp.zeros((_NPAD - _NCAND,), dtype=arr.dtype)])
        padded.append(flat)
    return padded


def _scatter_levels(level_ids, qs, vs, ms, acc, qb, vb, acc_words):
    wid = lax.axis_index("s") * 2 + lax.axis_index("c")

    def zero_body(i_, _):
        acc[pl.ds(i_ * 16, 16)] = jnp.zeros((16,), jnp.float32)
        return 0

    lax.fori_loop(0, acc_words // 16, zero_body, 0, unroll=8)

    for i, l in enumerate(level_ids):
        pltpu.sync_copy(qs[i], qb)
        pltpu.sync_copy(vs[i], vb)
        passes, rows = _SC_SPLIT[l]
        w0 = rows * _STR
        for pp in range(passes):
            base = (wid + pp * _NTILES) * w0

            def scat_body(i_, _, sign):
                qv = qb[pl.ds(i_ * 16, 16)]
                vv = vb[pl.ds(i_ * 16, 16)]
                loc = qv - base
                msk = (loc >= 0) & (loc < w0)
                loc = jnp.where(msk, loc, 0)
                plsc.addupdate_scatter(acc, [loc], sign * vv, mask=msk)
                return 0

            lax.fori_loop(0, _NPAD // 16,
                          functools.partial(scat_body, sign=1.0), 0, unroll=4)
            pltpu.sync_copy(acc.at[pl.ds(0, w0)], ms[i].at[pl.ds(base, w0)])
            if not (i == len(level_ids) - 1 and pp == passes - 1):
                lax.fori_loop(
                    0, _NPAD // 16,
                    functools.partial(scat_body, sign=-1.0), 0, unroll=4)


def _sc21_body(q1, v1, q2, v2, m1, m2, acc, qb, vb):
    _scatter_levels((1, 2), (q1, q2), (v1, v2), (m1, m2), acc, qb, vb,
                    _SC_SPLIT[1][1] * _STR)


def _sc0_body(q0, v0, m0, acc, qb, vb):
    _scatter_levels((0,), (q0,), (v0,), (m0,), acc, qb, vb,
                    _SC_SPLIT[0][1] * _STR)


def _sc_kernel(body, n_levels, acc_words):
    mesh = plsc.VectorSubcoreMesh(core_axis_name="c", subcore_axis_name="s")
    return functools.partial(
        pl.kernel,
        out_type=[
            jax.ShapeDtypeStruct((_B * H * W * _STR,), jnp.float32)
            for (_, H, W) in n_levels
        ],
        scratch_types=[
            pltpu.VMEM((acc_words,), jnp.float32),
            pltpu.VMEM((_NPAD,), jnp.int32),
            pltpu.VMEM((_NPAD,), jnp.float32),
        ],
        mesh=mesh,
        compiler_params=pltpu.CompilerParams(needs_layout_passes=False),
    )(body)


def _build_count_matrices(qv_list):
    q0, v0, q1, v1, q2, v2 = qv_list
    mt1, mt2 = _sc_kernel(
        _sc21_body, [_LVL[1], _LVL[2]],
        _SC_SPLIT[1][1] * _STR)(q1, v1, q2, v2)
    (mt0,) = _sc_kernel(
        _sc0_body, [_LVL[0]],
        _SC_SPLIT[0][1] * _STR)(q0, v0)
    return mt0, mt1, mt2


def _mm_step(feat_ref, mt_ref, acc, cacc):
    sg = jax.nn.sigmoid(feat_ref[0]).astype(jnp.bfloat16)
    mt = mt_ref[...]
    acc[...] += lax.dot_general(
        sg, mt.astype(jnp.bfloat16),
        dimension_numbers=(((1,), (0,)), ((), ())),
        preferred_element_type=jnp.float32,
    )
    cacc[...] += jnp.sum(mt, axis=0, keepdims=True)


def _mm_epilogue(acc, cacc, proto_ref, out_ref):
    cnt = cacc[...][:, :_NCLS]
    pks_t = acc[...][:, :_NCLS] / jnp.maximum(cnt, 1.0)
    g_t = proto_ref[...].T
    dots = jnp.sum(g_t * pks_t, axis=0, keepdims=True)
    nx = jnp.maximum(
        jnp.sqrt(jnp.sum(g_t * g_t, axis=0, keepdims=True) + 1e-12), 1e-8)
    ny = jnp.maximum(
        jnp.sqrt(jnp.sum(pks_t * pks_t, axis=0, keepdims=True) + 1e-12), 1e-8)
    aw = (dots / (nx * ny) + 1.0) * 0.5
    upd_t = aw * pks_t + (1.0 - aw) * g_t
    res_t = jnp.where(cnt > 0.0, upd_t, g_t)
    out_ref[...] = res_t.T


def _mm21_body(f1, m1, f2, m2, p1, p2, o1, o2, a1, c1, a2, c2):
    s = pl.program_id(0)

    @pl.when(s == 0)
    def _():
        for a, c in ((a1, c1), (a2, c2)):
            a[...] = jnp.zeros_like(a)
            c[...] = jnp.zeros_like(c)

    @pl.when(s < 8)
    def _():
        _mm_step(f1, m1, a1, c1)

    @pl.when(s == 7)
    def _():
        _mm_epilogue(a1, c1, p1, o1)

    @pl.when(s >= 8)
    def _():
        _mm_step(f2, m2, a2, c2)

    @pl.when(s == 15)
    def _():
        _mm_epilogue(a2, c2, p2, o2)


def _mm0_body(f0, m0, p0, o0, a0, c0):
    s = pl.program_id(0)

    @pl.when(s == 0)
    def _():
        a0[...] = jnp.zeros_like(a0)
        c0[...] = jnp.zeros_like(c0)

    _mm_step(f0, m0, a0, c0)

    @pl.when(s == 39)
    def _():
        _mm_epilogue(a0, c0, p0, o0)


def _proto_update_all(feats, mts, protos):
    f0 = jnp.reshape(feats[0], (_B, 128, 6400))
    f1 = jnp.reshape(feats[1], (_B, 256, 1600))
    f2 = jnp.reshape(feats[2], (_B, 512, 400))
    m0 = jnp.reshape(mts[0], (_B * 6400, _STR))
    m1 = jnp.reshape(mts[1], (_B * 1600, _STR))
    m2 = jnp.reshape(mts[2], (_B * 400, _STR))
    out1, out2 = pl.pallas_call(
        _mm21_body,
        grid=(16,),
        in_specs=[
            pl.BlockSpec((1, 256, 1600),
                         lambda s: (jnp.clip(s, 0, 7), 0, 0)),
            pl.BlockSpec((1600, _STR), lambda s: (jnp.clip(s, 0, 7), 0)),
            pl.BlockSpec((1, 512, 400),
                         lambda s: (jnp.clip(s - 8, 0, 7), 0, 0)),
            pl.BlockSpec((400, _STR), lambda s: (jnp.clip(s - 8, 0, 7), 0)),
            pl.BlockSpec((_NCLS, 256), lambda s: (0, 0)),
            pl.BlockSpec((_NCLS, 512), lambda s: (0, 0)),
        ],
        out_specs=[
            pl.BlockSpec((_NCLS, 256), lambda s: (0, 0)),
            pl.BlockSpec((_NCLS, 512), lambda s: (0, 0)),
        ],
        out_shape=[
            jax.ShapeDtypeStruct((_NCLS, 256), jnp.float32),
            jax.ShapeDtypeStruct((_NCLS, 512), jnp.float32),
        ],
        scratch_shapes=[
            pltpu.VMEM((256, _STR), jnp.float32),
            pltpu.VMEM((1, _STR), jnp.float32),
            pltpu.VMEM((512, _STR), jnp.float32),
            pltpu.VMEM((1, _STR), jnp.float32),
        ],
    )(f1, m1, f2, m2, protos[1], protos[2])
    out0 = pl.pallas_call(
        _mm0_body,
        grid=(40,),
        in_specs=[
            pl.BlockSpec((1, 128, 1280), lambda s: (s // 5, 0, s % 5)),
            pl.BlockSpec((1280, _STR), lambda s: (s, 0)),
            pl.BlockSpec((_NCLS, 128), lambda s: (0, 0)),
        ],
        out_specs=pl.BlockSpec((_NCLS, 128), lambda s: (0, 0)),
        out_shape=jax.ShapeDtypeStruct((_NCLS, 128), jnp.float32),
        scratch_shapes=[
            pltpu.VMEM((128, _STR), jnp.float32),
            pltpu.VMEM((1, _STR), jnp.float32),
        ],
    )(f0, m0, protos[0])
    return out0, out1, out2


def kernel(feat0, feat1, feat2, target, proto0, proto1, proto2):
    qv = _build_indices(target)
    mts = _build_count_matrices(qv)
    return _proto_update_all((feat0, feat1, feat2), mts,
                             (proto0, proto1, proto2))


# speedup vs baseline: 1.0084x; 1.0084x over previous
import functools

import jax
import jax.numpy as jnp
import numpy as np
from jax import lax
from jax.experimental import pallas as pl
from jax.experimental.pallas import tpu as pltpu
from jax.experimental.pallas import tpu_sc as plsc

_NCLS = 80
_ANCH = np.array(
    [[10., 13., 16., 30., 33., 23.],
     [30., 61., 62., 45., 59., 119.],
     [116., 90., 156., 198., 373., 326.]],
    dtype=np.float32,
).reshape(3, 3, 2)
_OFFS = [(0.0, 0.0), (0.5, 0.0), (0.0, 0.5), (-0.5, 0.0), (0.0, -0.5)]

_LVL = [(128, 80, 80), (256, 40, 40), (512, 20, 20)]
_B = 8
_NT = 300
_NCAND = 15 * _NT
_NPAD = 4512
_STR = 128

_SC_SPLIT = [(2, 800), (1, 400), (1, 100)]
_NTILES = 32


def _idx_body(tt_ref, q0, v0, q1, v1, q2, v2):
    tt = tt_ref[...]
    img, cls = tt[0:1], tt[1:2]
    x, y, w, h = tt[2:3], tt[3:4], tt[4:5], tt[5:6]
    b = img.astype(jnp.int32)
    c = cls.astype(jnp.int32)
    qrefs = (q0, q1, q2)
    vrefs = (v0, v1, v2)
    for l, (_, H, W) in enumerate(_LVL):
        gx, gy = x * W, y * H
        gw, gh = w * W, h * H
        jms = []
        for a in range(3):
            aw, ah = float(_ANCH[l, a, 0]), float(_ANCH[l, a, 1])
            rw, rh = gw / aw, gh / ah
            ratio = jnp.maximum(jnp.maximum(rw, 1.0 / rw),
                                jnp.maximum(rh, 1.0 / rh))
            jms.append(ratio < 4.0)
        gxi, gyi = W - gx, H - gy
        jj = ((gx % 1.0) < 0.5) & (gx > 1.0)
        kk = ((gy % 1.0) < 0.5) & (gy > 1.0)
        ll = ((gxi % 1.0) < 0.5) & (gxi > 1.0)
        mm = ((gyi % 1.0) < 0.5) & (gyi > 1.0)
        sels = [jnp.ones_like(jj), jj, kk, ll, mm]
        qrows, vrows = [], []
        for o in range(5):
            ox, oy = _OFFS[o]
            gi = jnp.clip((gx - ox).astype(jnp.int32), 0, W - 1)
            gj = jnp.clip((gy - oy).astype(jnp.int32), 0, H - 1)
            qo = (b * (H * W) + gj * W + gi) * _STR + c
            for a in range(3):
                qrows.append(qo)
                vrows.append((sels[o] & jms[a]).astype(jnp.float32))
        qrefs[l][...] = jnp.concatenate(qrows, axis=0)
        vrefs[l][...] = jnp.concatenate(vrows, axis=0)


def _build_indices(target):
    tt = jnp.transpose(target)
    outs = pl.pallas_call(
        _idx_body,
        out_shape=[
            s
            for _ in range(3)
            for s in (jax.ShapeDtypeStruct((15, _NT), jnp.int32),
                      jax.ShapeDtypeStruct((15, _NT), jnp.float32))
        ],
    )(tt)
    padded = []
    for arr in outs:
        flat = jnp.reshape(arr, (_NCAND,))
        flat = jnp.concatenate(
            [flat, jnp.zeros((_NPAD - _NCAND,), dtype=arr.dtype)])
        padded.append(flat)
    return padded


def _scatter_levels(level_ids, qs, vs, ms, acc, qb, vb, acc_words):
    wid = lax.axis_index("s") * 2 + lax.axis_index("c")
    for i, l in enumerate(level_ids):
        pltpu.sync_copy(qs[i], qb)
        pltpu.sync_copy(vs[i], vb)
        passes, rows = _SC_SPLIT[l]
        w0 = rows * _STR
        for pp in range(passes):
            base = (wid + pp * _NTILES) * w0

            def zero_body(i_, _):
                acc[pl.ds(i_ * 16, 16)] = jnp.zeros((16,), jnp.float32)
                return 0

            lax.fori_loop(0, w0 // 16, zero_body, 0, unroll=8)

            def scat_body(i_, _):
                qv = qb[pl.ds(i_ * 16, 16)]
                vv = vb[pl.ds(i_ * 16, 16)]
                loc = qv - base
                msk = (loc >= 0) & (loc < w0)
                loc = jnp.where(msk, loc, 0)
                plsc.addupdate_scatter(acc, [loc], vv, mask=msk)
                return 0

            lax.fori_loop(0, _NPAD // 16, scat_body, 0, unroll=4)
            pltpu.sync_copy(acc.at[pl.ds(0, w0)], ms[i].at[pl.ds(base, w0)])


def _sc21_body(q1, v1, q2, v2, m1, m2, acc, qb, vb):
    _scatter_levels((1, 2), (q1, q2), (v1, v2), (m1, m2), acc, qb, vb,
                    _SC_SPLIT[1][1] * _STR)


def _sc0_body(q0, v0, m0, acc, qb, vb):
    _scatter_levels((0,), (q0,), (v0,), (m0,), acc, qb, vb,
                    _SC_SPLIT[0][1] * _STR)


def _sc_kernel(body, n_levels, acc_words):
    mesh = plsc.VectorSubcoreMesh(core_axis_name="c", subcore_axis_name="s")
    return functools.partial(
        pl.kernel,
        out_type=[
            jax.ShapeDtypeStruct((_B * H * W * _STR,), jnp.float32)
            for (_, H, W) in n_levels
        ],
        scratch_types=[
            pltpu.VMEM((acc_words,), jnp.float32),
            pltpu.VMEM((_NPAD,), jnp.int32),
            pltpu.VMEM((_NPAD,), jnp.float32),
        ],
        mesh=mesh,
        compiler_params=pltpu.CompilerParams(needs_layout_passes=False),
    )(body)


def _build_count_matrices(qv_list):
    q0, v0, q1, v1, q2, v2 = qv_list
    mt1, mt2 = _sc_kernel(
        _sc21_body, [_LVL[1], _LVL[2]],
        _SC_SPLIT[1][1] * _STR)(q1, v1, q2, v2)
    (mt0,) = _sc_kernel(
        _sc0_body, [_LVL[0]],
        _SC_SPLIT[0][1] * _STR)(q0, v0)
    return mt0, mt1, mt2


def _mm_step(feat_ref, mt_ref, acc, cacc):
    sg = jax.nn.sigmoid(feat_ref[0]).astype(jnp.bfloat16)
    mt = mt_ref[...]
    acc[...] += lax.dot_general(
        sg, mt.astype(jnp.bfloat16),
        dimension_numbers=(((1,), (0,)), ((), ())),
        preferred_element_type=jnp.float32,
    )
    cacc[...] += jnp.sum(mt, axis=0, keepdims=True)


def _mm_epilogue(acc, cacc, proto_ref, out_ref):
    cnt = cacc[...][:, :_NCLS]
    pks_t = acc[...][:, :_NCLS] / jnp.maximum(cnt, 1.0)
    g_t = proto_ref[...].T
    dots = jnp.sum(g_t * pks_t, axis=0, keepdims=True)
    nx = jnp.maximum(
        jnp.sqrt(jnp.sum(g_t * g_t, axis=0, keepdims=True) + 1e-12), 1e-8)
    ny = jnp.maximum(
        jnp.sqrt(jnp.sum(pks_t * pks_t, axis=0, keepdims=True) + 1e-12), 1e-8)
    aw = (dots / (nx * ny) + 1.0) * 0.5
    upd_t = aw * pks_t + (1.0 - aw) * g_t
    res_t = jnp.where(cnt > 0.0, upd_t, g_t)
    out_ref[...] = res_t.T


def _mm21_body(f1, m1, f2, m2, p1, p2, o1, o2, a1, c1, a2, c2):
    s = pl.program_id(0)

    @pl.when(s == 0)
    def _():
        for a, c in ((a1, c1), (a2, c2)):
            a[...] = jnp.zeros_like(a)
            c[...] = jnp.zeros_like(c)

    @pl.when(s < 8)
    def _():
        _mm_step(f1, m1, a1, c1)

    @pl.when(s == 7)
    def _():
        _mm_epilogue(a1, c1, p1, o1)

    @pl.when(s >= 8)
    def _():
        _mm_step(f2, m2, a2, c2)

    @pl.when(s == 15)
    def _():
        _mm_epilogue(a2, c2, p2, o2)


def _mm0_body(f0, m0, p0, o0, a0, c0):
    s = pl.program_id(0)

    @pl.when(s == 0)
    def _():
        a0[...] = jnp.zeros_like(a0)
        c0[...] = jnp.zeros_like(c0)

    _mm_step(f0, m0, a0, c0)

    @pl.when(s == 39)
    def _():
        _mm_epilogue(a0, c0, p0, o0)


def _proto_update_all(feats, mts, protos):
    f0 = jnp.reshape(feats[0], (_B, 128, 6400))
    f1 = jnp.reshape(feats[1], (_B, 256, 1600))
    f2 = jnp.reshape(feats[2], (_B, 512, 400))
    m0 = jnp.reshape(mts[0], (_B * 6400, _STR))
    m1 = jnp.reshape(mts[1], (_B * 1600, _STR))
    m2 = jnp.reshape(mts[2], (_B * 400, _STR))
    out1, out2 = pl.pallas_call(
        _mm21_body,
        grid=(16,),
        in_specs=[
            pl.BlockSpec((1, 256, 1600),
                         lambda s: (jnp.clip(s, 0, 7), 0, 0)),
            pl.BlockSpec((1600, _STR), lambda s: (jnp.clip(s, 0, 7), 0)),
            pl.BlockSpec((1, 512, 400),
                         lambda s: (jnp.clip(s - 8, 0, 7), 0, 0)),
            pl.BlockSpec((400, _STR), lambda s: (jnp.clip(s - 8, 0, 7), 0)),
            pl.BlockSpec((_NCLS, 256), lambda s: (0, 0)),
            pl.BlockSpec((_NCLS, 512), lambda s: (0, 0)),
        ],
        out_specs=[
            pl.BlockSpec((_NCLS, 256), lambda s: (0, 0)),
            pl.BlockSpec((_NCLS, 512), lambda s: (0, 0)),
        ],
        out_shape=[
            jax.ShapeDtypeStruct((_NCLS, 256), jnp.float32),
            jax.ShapeDtypeStruct((_NCLS, 512), jnp.float32),
        ],
        scratch_shapes=[
            pltpu.VMEM((256, _STR), jnp.float32),
            pltpu.VMEM((1, _STR), jnp.float32),
            pltpu.VMEM((512, _STR), jnp.float32),
            pltpu.VMEM((1, _STR), jnp.float32),
        ],
    )(f1, m1, f2, m2, protos[1], protos[2])
    out0 = pl.pallas_call(
        _mm0_body,
        grid=(40,),
        in_specs=[
            pl.BlockSpec((1, 128, 1280), lambda s: (s // 5, 0, s % 5)),
            pl.BlockSpec((1280, _STR), lambda s: (s, 0)),
            pl.BlockSpec((_NCLS, 128), lambda s: (0, 0)),
        ],
        out_specs=pl.BlockSpec((_NCLS, 128), lambda s: (0, 0)),
        out_shape=jax.ShapeDtypeStruct((_NCLS, 128), jnp.float32),
        scratch_shapes=[
            pltpu.VMEM((128, _STR), jnp.float32),
            pltpu.VMEM((1, _STR), jnp.float32),
        ],
    )(f0, m0, protos[0])
    return out0, out1, out2


def kernel(feat0, feat1, feat2, target, proto0, proto1, proto2):
    qv = _build_indices(target)
    mts = _build_count_matrices(qv)
    return _proto_update_all((feat0, feat1, feat2), mts,
                             (proto0, proto1, proto2))


# speedup vs baseline: 1.1177x; 1.1083x over previous
import functools

import jax
import jax.numpy as jnp
import numpy as np
from jax import lax
from jax.experimental import pallas as pl
from jax.experimental.pallas import tpu as pltpu
from jax.experimental.pallas import tpu_sc as plsc

_NCLS = 80
_ANCH = np.array(
    [[10., 13., 16., 30., 33., 23.],
     [30., 61., 62., 45., 59., 119.],
     [116., 90., 156., 198., 373., 326.]],
    dtype=np.float32,
).reshape(3, 3, 2)
_OFFS = [(0.0, 0.0), (0.5, 0.0), (0.0, 0.5), (-0.5, 0.0), (0.0, -0.5)]

_LVL = [(128, 80, 80), (256, 40, 40), (512, 20, 20)]
_B = 8
_NT = 300
_NCAND = 15 * _NT
_NPAD = 4512
_STR = 128

_SC_SPLIT = [(2, 800), (1, 400), (1, 100)]
_NTILES = 32


def _idx_body(tt_ref, q0, v0, q1, v1, q2, v2):
    tt = tt_ref[...]
    img, cls = tt[0:1], tt[1:2]
    x, y, w, h = tt[2:3], tt[3:4], tt[4:5], tt[5:6]
    b = img.astype(jnp.int32)
    c = cls.astype(jnp.int32)
    qrefs = (q0, q1, q2)
    vrefs = (v0, v1, v2)
    for l, (_, H, W) in enumerate(_LVL):
        gx, gy = x * W, y * H
        gw, gh = w * W, h * H
        jms = []
        for a in range(3):
            aw, ah = float(_ANCH[l, a, 0]), float(_ANCH[l, a, 1])
            rw, rh = gw / aw, gh / ah
            ratio = jnp.maximum(jnp.maximum(rw, 1.0 / rw),
                                jnp.maximum(rh, 1.0 / rh))
            jms.append(ratio < 4.0)
        gxi, gyi = W - gx, H - gy
        jj = ((gx % 1.0) < 0.5) & (gx > 1.0)
        kk = ((gy % 1.0) < 0.5) & (gy > 1.0)
        ll = ((gxi % 1.0) < 0.5) & (gxi > 1.0)
        mm = ((gyi % 1.0) < 0.5) & (gyi > 1.0)
        sels = [jnp.ones_like(jj), jj, kk, ll, mm]
        qrows, vrows = [], []
        for o in range(5):
            ox, oy = _OFFS[o]
            gi = jnp.clip((gx - ox).astype(jnp.int32), 0, W - 1)
            gj = jnp.clip((gy - oy).astype(jnp.int32), 0, H - 1)
            qo = (b * (H * W) + gj * W + gi) * _STR + c
            for a in range(3):
                qrows.append(qo)
                vrows.append((sels[o] & jms[a]).astype(jnp.float32))
        qrefs[l][...] = jnp.concatenate(qrows, axis=0)
        vrefs[l][...] = jnp.concatenate(vrows, axis=0)


def _build_indices(target):
    tt = jnp.transpose(target)
    outs = pl.pallas_call(
        _idx_body,
        out_shape=[
            s
            for _ in range(3)
            for s in (jax.ShapeDtypeStruct((15, _NT), jnp.int32),
                      jax.ShapeDtypeStruct((15, _NT), jnp.float32))
        ],
    )(tt)
    padded = []
    for arr in outs:
        flat = jnp.reshape(arr, (_NCAND,))
        flat = jnp.concatenate(
            [flat, jnp.zeros((_NPAD - _NCAND,), dtype=arr.dtype)])
        padded.append(flat)
    return padded


def _scatter_levels(level_ids, qs, vs, ms, acc, qb, vb, acc_words):
    wid = lax.axis_index("s") * 2 + lax.axis_index("c")
    for i, l in enumerate(level_ids):
        pltpu.sync_copy(qs[i], qb)
        pltpu.sync_copy(vs[i], vb)
        passes, rows = _SC_SPLIT[l]
        w0 = rows * _STR
        for pp in range(passes):
            base = (wid + pp * _NTILES) * w0

            def zero_body(i_, _):
                acc[pl.ds(i_ * 16, 16)] = jnp.zeros((16,), jnp.float32)
                return 0

            lax.fori_loop(0, w0 // 16, zero_body, 0, unroll=8)

            def scat_body(i_, _):
                qv = qb[pl.ds(i_ * 16, 16)]
                vv = vb[pl.ds(i_ * 16, 16)]
                loc = qv - base
                msk = (loc >= 0) & (loc < w0)
                loc = jnp.where(msk, loc, 0)
                plsc.addupdate_scatter(acc, [loc], vv, mask=msk)
                return 0

            lax.fori_loop(0, _NPAD // 16, scat_body, 0, unroll=4)
            pltpu.sync_copy(acc.at[pl.ds(0, w0)], ms[i].at[pl.ds(base, w0)])


def _sc21_body(q1, v1, q2, v2, m1, m2, acc, qb, vb):
    _scatter_levels((1, 2), (q1, q2), (v1, v2), (m1, m2), acc, qb, vb,
                    _SC_SPLIT[1][1] * _STR)


def _sc0_body(q0, v0, m0, acc, qb, vb):
    _scatter_levels((0,), (q0,), (v0,), (m0,), acc, qb, vb,
                    _SC_SPLIT[0][1] * _STR)


def _sc_kernel(body, n_levels, acc_words):
    mesh = plsc.VectorSubcoreMesh(core_axis_name="c", subcore_axis_name="s")
    return functools.partial(
        pl.kernel,
        out_type=[
            jax.ShapeDtypeStruct((_B * H * W * _STR,), jnp.float32)
            for (_, H, W) in n_levels
        ],
        scratch_types=[
            pltpu.VMEM((acc_words,), jnp.float32),
            pltpu.VMEM((_NPAD,), jnp.int32),
            pltpu.VMEM((_NPAD,), jnp.float32),
        ],
        mesh=mesh,
        compiler_params=pltpu.CompilerParams(needs_layout_passes=False),
    )(body)


def _build_count_matrices(qv_list):
    q0, v0, q1, v1, q2, v2 = qv_list
    mt1, mt2 = _sc_kernel(
        _sc21_body, [_LVL[1], _LVL[2]],
        _SC_SPLIT[1][1] * _STR)(q1, v1, q2, v2)
    (mt0,) = _sc_kernel(
        _sc0_body, [_LVL[0]],
        _SC_SPLIT[0][1] * _STR)(q0, v0)
    return mt0, mt1, mt2


def _mm_step(feat_ref, mt_ref, acc, cacc):
    sg = jax.nn.sigmoid(feat_ref[0]).astype(jnp.bfloat16)
    mt = mt_ref[...]
    acc[...] += lax.dot_general(
        sg, mt.astype(jnp.bfloat16),
        dimension_numbers=(((1,), (0,)), ((), ())),
        preferred_element_type=jnp.float32,
    )
    cacc[...] += jnp.sum(mt, axis=0, keepdims=True)


def _mm_epilogue(acc, cacc, proto_ref, out_ref):
    cnt = cacc[...][:, :_NCLS]
    pks_t = acc[...][:, :_NCLS] / jnp.maximum(cnt, 1.0)
    g_t = proto_ref[...].T
    dots = jnp.sum(g_t * pks_t, axis=0, keepdims=True)
    nx = jnp.maximum(
        jnp.sqrt(jnp.sum(g_t * g_t, axis=0, keepdims=True) + 1e-12), 1e-8)
    ny = jnp.maximum(
        jnp.sqrt(jnp.sum(pks_t * pks_t, axis=0, keepdims=True) + 1e-12), 1e-8)
    aw = (dots / (nx * ny) + 1.0) * 0.5
    upd_t = aw * pks_t + (1.0 - aw) * g_t
    res_t = jnp.where(cnt > 0.0, upd_t, g_t)
    out_ref[...] = res_t.T


def _mm21_body(f1, m1, f2, m2, p1, p2, o1, o2, a1, c1, a2, c2):
    s = pl.program_id(0)

    @pl.when(s == 0)
    def _():
        for a, c in ((a1, c1), (a2, c2)):
            a[...] = jnp.zeros_like(a)
            c[...] = jnp.zeros_like(c)

    @pl.when(s < 8)
    def _():
        _mm_step(f1, m1, a1, c1)

    @pl.when(s == 7)
    def _():
        _mm_epilogue(a1, c1, p1, o1)

    @pl.when(s >= 8)
    def _():
        _mm_step(f2, m2, a2, c2)

    @pl.when(s == 15)
    def _():
        _mm_epilogue(a2, c2, p2, o2)


def _mm0_body(f0, m0, p0, o0, a0, c0):
    s = pl.program_id(0)

    @pl.when(s == 0)
    def _():
        a0[...] = jnp.zeros_like(a0)
        c0[...] = jnp.zeros_like(c0)

    _mm_step(f0, m0, a0, c0)

    @pl.when(s == 15)
    def _():
        _mm_epilogue(a0, c0, p0, o0)


def _proto_update_all(feats, mts, protos):
    f0 = jnp.reshape(feats[0], (_B, 128, 6400))
    f1 = jnp.reshape(feats[1], (_B, 256, 1600))
    f2 = jnp.reshape(feats[2], (_B, 512, 400))
    m0 = jnp.reshape(mts[0], (_B * 6400, _STR))
    m1 = jnp.reshape(mts[1], (_B * 1600, _STR))
    m2 = jnp.reshape(mts[2], (_B * 400, _STR))
    out1, out2 = pl.pallas_call(
        _mm21_body,
        grid=(16,),
        in_specs=[
            pl.BlockSpec((1, 256, 1600),
                         lambda s: (jnp.clip(s, 0, 7), 0, 0)),
            pl.BlockSpec((1600, _STR), lambda s: (jnp.clip(s, 0, 7), 0)),
            pl.BlockSpec((1, 512, 400),
                         lambda s: (jnp.clip(s - 8, 0, 7), 0, 0)),
            pl.BlockSpec((400, _STR), lambda s: (jnp.clip(s - 8, 0, 7), 0)),
            pl.BlockSpec((_NCLS, 256), lambda s: (0, 0)),
            pl.BlockSpec((_NCLS, 512), lambda s: (0, 0)),
        ],
        out_specs=[
            pl.BlockSpec((_NCLS, 256), lambda s: (0, 0)),
            pl.BlockSpec((_NCLS, 512), lambda s: (0, 0)),
        ],
        out_shape=[
            jax.ShapeDtypeStruct((_NCLS, 256), jnp.float32),
            jax.ShapeDtypeStruct((_NCLS, 512), jnp.float32),
        ],
        scratch_shapes=[
            pltpu.VMEM((256, _STR), jnp.float32),
            pltpu.VMEM((1, _STR), jnp.float32),
            pltpu.VMEM((512, _STR), jnp.float32),
            pltpu.VMEM((1, _STR), jnp.float32),
        ],
    )(f1, m1, f2, m2, protos[1], protos[2])
    out0 = pl.pallas_call(
        _mm0_body,
        grid=(16,),
        in_specs=[
            pl.BlockSpec((1, 128, 3200), lambda s: (s // 2, 0, s % 2)),
            pl.BlockSpec((3200, _STR), lambda s: (s, 0)),
            pl.BlockSpec((_NCLS, 128), lambda s: (0, 0)),
        ],
        out_specs=pl.BlockSpec((_NCLS, 128), lambda s: (0, 0)),
        out_shape=jax.ShapeDtypeStruct((_NCLS, 128), jnp.float32),
        scratch_shapes=[
            pltpu.VMEM((128, _STR), jnp.float32),
            pltpu.VMEM((1, _STR), jnp.float32),
        ],
    )(f0, m0, protos[0])
    return out0, out1, out2


def kernel(feat0, feat1, feat2, target, proto0, proto1, proto2):
    qv = _build_indices(target)
    mts = _build_count_matrices(qv)
    return _proto_update_all((feat0, feat1, feat2), mts,
                             (proto0, proto1, proto2))
